# baseline (speedup 1.0000x reference)
import functools
import math

import jax
import jax.numpy as jnp
from jax.experimental import pallas as pl
from jax.experimental.pallas import tpu as pltpu

D_IN = 64
D_OUT = 64

_ROWS = 1
_NS = 6
_AHEAD = 5

_TB = 4
_TL = 8192


def _manual_kernel(x_hbm, w_ref, b_ref, o_hbm, in_buf, out_buf, in_sem, out_sem,
                   *, nch):
    w = w_ref[...]
    bb = b_ref[...]

    def in_cp(c, s):
        return pltpu.make_async_copy(
            x_hbm.at[pl.ds(c * _ROWS, _ROWS)], in_buf.at[s], in_sem.at[s])

    def out_cp(c, s):
        return pltpu.make_async_copy(
            out_buf.at[s], o_hbm.at[pl.ds(c * _ROWS, _ROWS)], out_sem.at[s])

    for c in range(min(_AHEAD, nch)):
        in_cp(c, c % _NS).start()

    def step(c, carry):
        @pl.when(c + _AHEAD < nch)
        def _():
            cn = c + _AHEAD
            in_cp(cn, jax.lax.rem(cn, _NS)).start()
        s = jax.lax.rem(c, _NS)
        in_cp(c, s).wait()

        @pl.when(c >= _NS)
        def _():
            out_cp(c - _NS, s).wait()
        for t in range(_ROWS):
            out_buf[s, t] = (
                jnp.dot(w, in_buf[s, t], preferred_element_type=jnp.float32) + bb)
        out_cp(c, s).start()
        return carry

    jax.lax.fori_loop(0, nch, step, 0, unroll=False)

    for c in range(max(nch - _NS, 0), nch):
        out_cp(c, c % _NS).wait()


def _manual_channels_first(xt, w, b):
    B, C, L = xt.shape
    nch = B // _ROWS
    b_col = b.reshape(C, 1)
    return pl.pallas_call(
        functools.partial(_manual_kernel, nch=nch),
        out_shape=jax.ShapeDtypeStruct((B, C, L), jnp.float32),
        in_specs=[
            pl.BlockSpec(memory_space=pl.ANY),
            pl.BlockSpec(memory_space=pltpu.VMEM),
            pl.BlockSpec(memory_space=pltpu.VMEM),
        ],
        out_specs=pl.BlockSpec(memory_space=pl.ANY),
        scratch_shapes=[
            pltpu.VMEM((_NS, _ROWS, C, L), jnp.float32),
            pltpu.VMEM((_NS, _ROWS, C, L), jnp.float32),
            pltpu.SemaphoreType.DMA((_NS,)),
            pltpu.SemaphoreType.DMA((_NS,)),
        ],
    )(xt, w, b_col)


def _emitter_cf_kernel(x_ref, w_ref, b_ref, o_ref):
    for t in range(x_ref.shape[0]):
        acc = jnp.dot(w_ref[...], x_ref[t], preferred_element_type=jnp.float32)
        o_ref[t] = acc + b_ref[...]


def _emitter_channels_first(xt, w, b):
    B, C, L = xt.shape
    b_col = b.reshape(D_OUT, 1)
    tl = L if L <= _TL else _TL
    tb = _TB if B % _TB == 0 else 1
    grid = (B // tb, pl.cdiv(L, tl))
    cost = pl.CostEstimate(
        flops=2 * B * L * D_IN * D_OUT,
        transcendentals=0,
        bytes_accessed=2 * B * C * L * 4 + D_IN * D_OUT * 4 + D_OUT * 4,
    )
    return pl.pallas_call(
        _emitter_cf_kernel,
        out_shape=jax.ShapeDtypeStruct((B, D_OUT, L), jnp.float32),
        grid=grid,
        in_specs=[
            pl.BlockSpec((tb, D_IN, tl), lambda bi, li: (bi, 0, li)),
            pl.BlockSpec((D_OUT, D_IN), lambda bi, li: (0, 0)),
            pl.BlockSpec((D_OUT, 1), lambda bi, li: (0, 0)),
        ],
        out_specs=pl.BlockSpec((tb, D_OUT, tl), lambda bi, li: (bi, 0, li)),
        compiler_params=pltpu.CompilerParams(
            dimension_semantics=("parallel", "parallel"),
        ),
        cost_estimate=cost,
    )(xt, w, b_col)


def _linear_channels_first(xt, w, b):
    B, C, L = xt.shape
    if B % _ROWS == 0 and B // _ROWS >= _AHEAD and _NS * 2 * _ROWS * C * L * 4 <= 40 * 2**20:
        return _manual_channels_first(xt, w, b)
    return _emitter_channels_first(xt, w, b)


def kernel(x, w, b):
    if x.ndim == 3:
        xt = jnp.transpose(x, (1, 2, 0))
        yt = _linear_channels_first(xt, w, b)
        return jnp.transpose(yt, (2, 0, 1))

    lead = x.shape[:-1]
    M = math.prod(lead) if lead else 1
    x2d = x.reshape(M, D_IN)
    m_pad = -M % 8
    if m_pad:
        x2d = jnp.pad(x2d, ((0, m_pad), (0, 0)))
    xt = jnp.transpose(x2d, (1, 0)).reshape(1, D_IN, M + m_pad)
    yt = _emitter_channels_first(xt, w, b)
    y2d = jnp.transpose(yt[0], (1, 0))
    if m_pad:
        y2d = y2d[:M]
    return y2d.reshape(*lead, D_OUT)


# speedup vs baseline: 1.0015x; 1.0015x over previous
import functools
import math

import jax
import jax.numpy as jnp
from jax.experimental import pallas as pl
from jax.experimental.pallas import tpu as pltpu

D_IN = 64
D_OUT = 64

_ROWS = 2
_NS = 5
_AHEAD = 4

_TB = 4
_TL = 8192


def _manual_kernel(x_hbm, w_ref, b_ref, o_hbm, in_buf, out_buf, in_sem, out_sem,
                   *, nch):
    w = w_ref[...]
    bb = b_ref[...]

    def in_cp(c, s):
        return pltpu.make_async_copy(
            x_hbm.at[pl.ds(c * _ROWS, _ROWS)], in_buf.at[s], in_sem.at[s])

    def out_cp(c, s):
        return pltpu.make_async_copy(
            out_buf.at[s], o_hbm.at[pl.ds(c * _ROWS, _ROWS)], out_sem.at[s])

    for c in range(min(_AHEAD, nch)):
        in_cp(c, c % _NS).start()

    def step(c, carry):
        @pl.when(c + _AHEAD < nch)
        def _():
            cn = c + _AHEAD
            in_cp(cn, jax.lax.rem(cn, _NS)).start()
        s = jax.lax.rem(c, _NS)
        in_cp(c, s).wait()

        @pl.when(c >= _NS)
        def _():
            out_cp(c - _NS, s).wait()
        for t in range(_ROWS):
            out_buf[s, t] = (
                jnp.dot(w, in_buf[s, t], preferred_element_type=jnp.float32) + bb)
        out_cp(c, s).start()
        return carry

    jax.lax.fori_loop(0, nch, step, 0, unroll=False)

    for c in range(max(nch - _NS, 0), nch):
        out_cp(c, c % _NS).wait()


def _manual_channels_first(xt, w, b):
    B, C, L = xt.shape
    nch = B // _ROWS
    b_col = b.reshape(C, 1)
    return pl.pallas_call(
        functools.partial(_manual_kernel, nch=nch),
        out_shape=jax.ShapeDtypeStruct((B, C, L), jnp.float32),
        in_specs=[
            pl.BlockSpec(memory_space=pl.ANY),
            pl.BlockSpec(memory_space=pltpu.VMEM),
            pl.BlockSpec(memory_space=pltpu.VMEM),
        ],
        out_specs=pl.BlockSpec(memory_space=pl.ANY),
        scratch_shapes=[
            pltpu.VMEM((_NS, _ROWS, C, L), jnp.float32),
            pltpu.VMEM((_NS, _ROWS, C, L), jnp.float32),
            pltpu.SemaphoreType.DMA((_NS,)),
            pltpu.SemaphoreType.DMA((_NS,)),
        ],
    )(xt, w, b_col)


def _emitter_cf_kernel(x_ref, w_ref, b_ref, o_ref):
    for t in range(x_ref.shape[0]):
        acc = jnp.dot(w_ref[...], x_ref[t], preferred_element_type=jnp.float32)
        o_ref[t] = acc + b_ref[...]


def _emitter_channels_first(xt, w, b):
    B, C, L = xt.shape
    b_col = b.reshape(D_OUT, 1)
    tl = L if L <= _TL else _TL
    tb = _TB if B % _TB == 0 else 1
    grid = (B // tb, pl.cdiv(L, tl))
    cost = pl.CostEstimate(
        flops=2 * B * L * D_IN * D_OUT,
        transcendentals=0,
        bytes_accessed=2 * B * C * L * 4 + D_IN * D_OUT * 4 + D_OUT * 4,
    )
    return pl.pallas_call(
        _emitter_cf_kernel,
        out_shape=jax.ShapeDtypeStruct((B, D_OUT, L), jnp.float32),
        grid=grid,
        in_specs=[
            pl.BlockSpec((tb, D_IN, tl), lambda bi, li: (bi, 0, li)),
            pl.BlockSpec((D_OUT, D_IN), lambda bi, li: (0, 0)),
            pl.BlockSpec((D_OUT, 1), lambda bi, li: (0, 0)),
        ],
        out_specs=pl.BlockSpec((tb, D_OUT, tl), lambda bi, li: (bi, 0, li)),
        compiler_params=pltpu.CompilerParams(
            dimension_semantics=("parallel", "parallel"),
        ),
        cost_estimate=cost,
    )(xt, w, b_col)


def _linear_channels_first(xt, w, b):
    B, C, L = xt.shape
    if B % _ROWS == 0 and B // _ROWS >= _AHEAD and _NS * 2 * _ROWS * C * L * 4 <= 40 * 2**20:
        return _manual_channels_first(xt, w, b)
    return _emitter_channels_first(xt, w, b)


def kernel(x, w, b):
    if x.ndim == 3:
        xt = jnp.transpose(x, (1, 2, 0))
        yt = _linear_channels_first(xt, w, b)
        return jnp.transpose(yt, (2, 0, 1))

    lead = x.shape[:-1]
    M = math.prod(lead) if lead else 1
    x2d = x.reshape(M, D_IN)
    m_pad = -M % 8
    if m_pad:
        x2d = jnp.pad(x2d, ((0, m_pad), (0, 0)))
    xt = jnp.transpose(x2d, (1, 0)).reshape(1, D_IN, M + m_pad)
    yt = _emitter_channels_first(xt, w, b)
    y2d = jnp.transpose(yt[0], (1, 0))
    if m_pad:
        y2d = y2d[:M]
    return y2d.reshape(*lead, D_OUT)


# speedup vs baseline: 1.0029x; 1.0013x over previous
import functools
import math

import jax
import jax.numpy as jnp
from jax.experimental import pallas as pl
from jax.experimental.pallas import tpu as pltpu

D_IN = 64
D_OUT = 64

_ROWS = 2
_NS = 4
_AHEAD = 3

_TB = 4
_TL = 8192


def _manual_kernel(x_hbm, w_ref, b_ref, o_hbm, in_buf, out_buf, in_sem, out_sem,
                   *, nch):
    w = w_ref[...]
    bb = b_ref[...]

    def in_cp(c, s):
        return pltpu.make_async_copy(
            x_hbm.at[pl.ds(c * _ROWS, _ROWS)], in_buf.at[s], in_sem.at[s])

    def out_cp(c, s):
        return pltpu.make_async_copy(
            out_buf.at[s], o_hbm.at[pl.ds(c * _ROWS, _ROWS)], out_sem.at[s])

    for c in range(min(_AHEAD, nch)):
        in_cp(c, c % _NS).start()

    def step(c, carry):
        @pl.when(c + _AHEAD < nch)
        def _():
            cn = c + _AHEAD
            in_cp(cn, jax.lax.rem(cn, _NS)).start()
        s = jax.lax.rem(c, _NS)
        in_cp(c, s).wait()

        @pl.when(c >= _NS)
        def _():
            out_cp(c - _NS, s).wait()
        for t in range(_ROWS):
            out_buf[s, t] = (
                jnp.dot(w, in_buf[s, t], preferred_element_type=jnp.float32) + bb)
        out_cp(c, s).start()
        return carry

    jax.lax.fori_loop(0, nch, step, 0, unroll=False)

    for c in range(max(nch - _NS, 0), nch):
        out_cp(c, c % _NS).wait()


def _manual_channels_first(xt, w, b):
    B, C, L = xt.shape
    nch = B // _ROWS
    b_col = b.reshape(C, 1)
    return pl.pallas_call(
        functools.partial(_manual_kernel, nch=nch),
        out_shape=jax.ShapeDtypeStruct((B, C, L), jnp.float32),
        in_specs=[
            pl.BlockSpec(memory_space=pl.ANY),
            pl.BlockSpec(memory_space=pltpu.VMEM),
            pl.BlockSpec(memory_space=pltpu.VMEM),
        ],
        out_specs=pl.BlockSpec(memory_space=pl.ANY),
        scratch_shapes=[
            pltpu.VMEM((_NS, _ROWS, C, L), jnp.float32),
            pltpu.VMEM((_NS, _ROWS, C, L), jnp.float32),
            pltpu.SemaphoreType.DMA((_NS,)),
            pltpu.SemaphoreType.DMA((_NS,)),
        ],
    )(xt, w, b_col)


def _emitter_cf_kernel(x_ref, w_ref, b_ref, o_ref):
    for t in range(x_ref.shape[0]):
        acc = jnp.dot(w_ref[...], x_ref[t], preferred_element_type=jnp.float32)
        o_ref[t] = acc + b_ref[...]


def _emitter_channels_first(xt, w, b):
    B, C, L = xt.shape
    b_col = b.reshape(D_OUT, 1)
    tl = L if L <= _TL else _TL
    tb = _TB if B % _TB == 0 else 1
    grid = (B // tb, pl.cdiv(L, tl))
    cost = pl.CostEstimate(
        flops=2 * B * L * D_IN * D_OUT,
        transcendentals=0,
        bytes_accessed=2 * B * C * L * 4 + D_IN * D_OUT * 4 + D_OUT * 4,
    )
    return pl.pallas_call(
        _emitter_cf_kernel,
        out_shape=jax.ShapeDtypeStruct((B, D_OUT, L), jnp.float32),
        grid=grid,
        in_specs=[
            pl.BlockSpec((tb, D_IN, tl), lambda bi, li: (bi, 0, li)),
            pl.BlockSpec((D_OUT, D_IN), lambda bi, li: (0, 0)),
            pl.BlockSpec((D_OUT, 1), lambda bi, li: (0, 0)),
        ],
        out_specs=pl.BlockSpec((tb, D_OUT, tl), lambda bi, li: (bi, 0, li)),
        compiler_params=pltpu.CompilerParams(
            dimension_semantics=("parallel", "parallel"),
        ),
        cost_estimate=cost,
    )(xt, w, b_col)


def _linear_channels_first(xt, w, b):
    B, C, L = xt.shape
    if B % _ROWS == 0 and B // _ROWS >= _AHEAD and _NS * 2 * _ROWS * C * L * 4 <= 40 * 2**20:
        return _manual_channels_first(xt, w, b)
    return _emitter_channels_first(xt, w, b)


def kernel(x, w, b):
    if x.ndim == 3:
        xt = jnp.transpose(x, (1, 2, 0))
        yt = _linear_channels_first(xt, w, b)
        return jnp.transpose(yt, (2, 0, 1))

    lead = x.shape[:-1]
    M = math.prod(lead) if lead else 1
    x2d = x.reshape(M, D_IN)
    m_pad = -M % 8
    if m_pad:
        x2d = jnp.pad(x2d, ((0, m_pad), (0, 0)))
    xt = jnp.transpose(x2d, (1, 0)).reshape(1, D_IN, M + m_pad)
    yt = _emitter_channels_first(xt, w, b)
    y2d = jnp.transpose(yt[0], (1, 0))
    if m_pad:
        y2d = y2d[:M]
    return y2d.reshape(*lead, D_OUT)
